# Initial kernel scaffold; baseline (speedup 1.0000x reference)
import jax
import jax.numpy as jnp
from jax.experimental import pallas as pl


def kernel(x, router_w, router_b, W1, W2, W3):
    raise NotImplementedError("write your pallas kernel here")


# speedup vs baseline: 3.1350x; 3.1350x over previous
import functools

import jax
import jax.numpy as jnp
from jax import lax
from jax.experimental import pallas as pl
from jax.experimental.pallas import tpu as pltpu
from jax.experimental.pallas import tpu_sc as plsc

BM = 256
BM2 = 256


def _router_body(x_ref, rw_ref, rb_ref,
                 pos_ref, w_ref, eob_ref, totb_ref, aux_ref,
                 eidx_s, rank_s, w_s, cnt_ref, off_s, s_ref,
                 *, E, NT, BM, BM2, NBLK):
    p = pl.program_id(0)
    i = pl.program_id(1)
    sl = pl.ds(i * BM, BM)

    @pl.when(p == 0)
    def _():
        xt = x_ref[...]
        rw = rw_ref[...]
        logits = lax.dot_general(xt, rw, (((1,), (1,)), ((), ())),
                                 preferred_element_type=jnp.float32) + rb_ref[...]
        nrm = jnp.sqrt(jnp.sum(logits * logits, axis=-1, keepdims=True))
        rl = logits / jnp.maximum(nrm, 1e-12)
        m = jnp.max(rl, axis=-1, keepdims=True)
        ex = jnp.exp(rl - m)
        probs = ex / jnp.sum(ex, axis=-1, keepdims=True)

        lane = lax.broadcasted_iota(jnp.int32, probs.shape, 1)
        m1 = jnp.max(probs, axis=-1, keepdims=True)
        i1 = jnp.min(jnp.where(probs == m1, lane, E), axis=-1, keepdims=True)
        pr2 = jnp.where(lane == i1, -jnp.inf, probs)
        m2 = jnp.max(pr2, axis=-1, keepdims=True)
        i2 = jnp.min(jnp.where(pr2 == m2, lane, E), axis=-1, keepdims=True)

        oh0 = (lane == i1).astype(jnp.float32)
        oh1 = (lane == i2).astype(jnp.float32)
        r = lax.broadcasted_iota(jnp.int32, (BM, BM), 0)
        c = lax.broadcasted_iota(jnp.int32, (BM, BM), 1)
        tri = (r > c).astype(jnp.float32)
        excl0 = lax.dot_general(tri, oh0, (((1,), (0,)), ((), ())),
                                preferred_element_type=jnp.float32)
        tot0 = jnp.sum(oh0, axis=0, keepdims=True)
        excl1 = lax.dot_general(tri, oh1, (((1,), (0,)), ((), ())),
                                preferred_element_type=jnp.float32) + tot0
        tot1 = jnp.sum(oh1, axis=0, keepdims=True)

        base = jnp.where(i == 0, jnp.zeros((1, E), jnp.float32), cnt_ref[...])
        rank0 = jnp.sum(oh0 * (excl0 + base), axis=1, keepdims=True)
        rank1 = jnp.sum(oh1 * (excl1 + base), axis=1, keepdims=True)
        newcnt = base + tot0 + tot1
        cnt_ref[...] = newcnt

        eidx_s[sl, :] = jnp.concatenate([i1, i2], axis=1)
        rank_s[sl, :] = jnp.concatenate([rank0, rank1], axis=1).astype(jnp.int32)
        w_s[sl, :] = jnp.concatenate([m1, m2], axis=1)

        partial = jnp.sum((1.0 / E - probs) ** 2)
        tot = jnp.where(i == 0, 0.0, s_ref[0]) + partial
        s_ref[0] = tot

        @pl.when(i == NT - 1)
        def _():
            cnt_i = newcnt.astype(jnp.int32)
            ntb = (cnt_i + (BM2 - 1)) // BM2
            ntb_f = ntb.astype(jnp.float32)
            rr = lax.broadcasted_iota(jnp.int32, (E, E), 0)
            cc = lax.broadcasted_iota(jnp.int32, (E, E), 1)
            ustr = (rr < cc).astype(jnp.float32)
            offb = lax.dot_general(ntb_f, ustr, (((1,), (0,)), ((), ())),
                                   preferred_element_type=jnp.float32)
            offb_i = offb.astype(jnp.int32)
            off_s[...] = offb_i * BM2
            blkio = lax.broadcasted_iota(jnp.int32, (1, NBLK), 1)
            eob = jnp.full((1, NBLK), -1, jnp.int32)
            for e in range(E):
                eob = eob + jnp.where(blkio >= offb_i[0:1, e:e + 1], 1, 0)
            eob_ref[...] = eob
            totb_ref[...] = offb_i[0:1, E - 1:E] + ntb[0:1, E - 1:E]
            aux_ref[...] = jnp.full((1, 1), tot, dtype=jnp.float32)

    @pl.when(p == 1)
    def _():
        ei = eidx_s[sl, :]
        rk = rank_s[sl, :]
        po = jnp.zeros(ei.shape, jnp.int32)
        for e in range(E):
            po = po + jnp.where(ei == e, 1, 0) * off_s[0:1, e:e + 1]
        pos_ref[...] = po + rk
        w_ref[...] = w_s[sl, :]


def _router_call(x2d, router_w, rb2, E):
    T, d = x2d.shape
    NT = T // BM
    NBLK = 2 * T // BM2 + E
    body = functools.partial(_router_body, E=E, NT=NT, BM=BM, BM2=BM2,
                             NBLK=NBLK)
    return pl.pallas_call(
        body,
        grid=(2, NT),
        in_specs=[
            pl.BlockSpec((BM, d), lambda p, i: (jnp.where(p == 0, i, NT - 1), 0)),
            pl.BlockSpec((E, d), lambda p, i: (0, 0)),
            pl.BlockSpec((1, E), lambda p, i: (0, 0)),
        ],
        out_specs=[
            pl.BlockSpec((BM, 2), lambda p, i: (jnp.where(p == 1, i, 0), 0)),
            pl.BlockSpec((BM, 2), lambda p, i: (jnp.where(p == 1, i, 0), 0)),
            pl.BlockSpec((1, NBLK), lambda p, i: (0, 0)),
            pl.BlockSpec((1, 1), lambda p, i: (0, 0)),
            pl.BlockSpec((1, 1), lambda p, i: (0, 0)),
        ],
        out_shape=[
            jax.ShapeDtypeStruct((T, 2), jnp.int32),
            jax.ShapeDtypeStruct((T, 2), jnp.float32),
            jax.ShapeDtypeStruct((1, NBLK), jnp.int32),
            jax.ShapeDtypeStruct((1, 1), jnp.int32),
            jax.ShapeDtypeStruct((1, 1), jnp.float32),
        ],
        scratch_shapes=[
            pltpu.VMEM((T, 2), jnp.int32),
            pltpu.VMEM((T, 2), jnp.int32),
            pltpu.VMEM((T, 2), jnp.float32),
            pltpu.VMEM((1, E), jnp.float32),
            pltpu.VMEM((1, E), jnp.int32),
            pltpu.SMEM((1,), jnp.float32),
        ],
        compiler_params=pltpu.CompilerParams(
            dimension_semantics=("arbitrary", "arbitrary")),
    )(x2d, router_w, rb2)


def _dispatch_sc(x2d, pos_s, NPAD):
    T, d = x2d.shape
    P = pos_s.shape[0]
    info = plsc.get_sparse_core_info()
    NC, NS = info.num_cores, info.num_subcores
    NW = NC * NS
    CP = P // NW
    mesh = plsc.VectorSubcoreMesh(core_axis_name="c", subcore_axis_name="s")

    @functools.partial(
        pl.kernel, mesh=mesh,
        out_type=jax.ShapeDtypeStruct((NPAD, d), jnp.float32),
        scratch_types=[
            pltpu.VMEM((CP,), jnp.int32),
            pltpu.VMEM((CP, d), jnp.float32),
            pltpu.SemaphoreType.DMA,
        ])
    def disp(x_hbm, pos_hbm, xs_hbm, pos_v, rows_v, sem):
        wid = lax.axis_index("s") * NC + lax.axis_index("c")
        base = wid * CP
        tok = base - jnp.where(base >= T, T, 0)
        pltpu.sync_copy(pos_hbm.at[pl.ds(base, CP)], pos_v)
        pltpu.sync_copy(x_hbm.at[pl.ds(tok, CP)], rows_v)
        pltpu.async_copy(rows_v, xs_hbm.at[pos_v], sem).wait()

    return disp(x2d, pos_s)


def _ffn_body(eob_ref, totb_ref, xs_ref, w1_ref, w2_ref, w3_ref, ys_ref):
    t = pl.program_id(0)

    @pl.when(t < totb_ref[0])
    def _():
        xt = xs_ref[...]
        w1 = w1_ref[0]
        w3 = w3_ref[0]
        w2 = w2_ref[0]
        dn = (((1,), (1,)), ((), ()))
        h = jnp.sin(lax.dot_general(xt, w1, dn,
                                    preferred_element_type=jnp.float32))
        h = h * lax.dot_general(xt, w3, dn,
                                preferred_element_type=jnp.float32)
        ys_ref[...] = lax.dot_general(h, w2, dn,
                                      preferred_element_type=jnp.float32)


def _ffn_call(xs, W1, W2, W3, eob, totb, NBLK):
    NPAD, d = xs.shape
    E, H, _ = W1.shape

    def xs_map(t, eob_ref, totb_ref):
        return (jnp.minimum(t, totb_ref[0] - 1), 0)

    def w_map(t, eob_ref, totb_ref):
        return (eob_ref[jnp.minimum(t, totb_ref[0] - 1)], 0, 0)

    grid_spec = pltpu.PrefetchScalarGridSpec(
        num_scalar_prefetch=2,
        grid=(NBLK,),
        in_specs=[
            pl.BlockSpec((BM2, d), xs_map),
            pl.BlockSpec((1, H, d), w_map),
            pl.BlockSpec((1, d, H), w_map),
            pl.BlockSpec((1, H, d), w_map),
        ],
        out_specs=pl.BlockSpec((BM2, d), xs_map),
    )
    return pl.pallas_call(
        _ffn_body,
        grid_spec=grid_spec,
        out_shape=jax.ShapeDtypeStruct((NPAD, d), jnp.float32),
        compiler_params=pltpu.CompilerParams(
            dimension_semantics=("arbitrary",)),
    )(eob, totb, xs, W1, W2, W3)


def _gather_sc(ys, pos):
    NPAD, d = ys.shape
    P = pos.shape[0]
    info = plsc.get_sparse_core_info()
    NC, NS = info.num_cores, info.num_subcores
    NW = NC * NS
    CP = P // NW
    mesh = plsc.VectorSubcoreMesh(core_axis_name="c", subcore_axis_name="s")

    @functools.partial(
        pl.kernel, mesh=mesh,
        out_type=jax.ShapeDtypeStruct((P, d), jnp.float32),
        scratch_types=[
            pltpu.VMEM((CP,), jnp.int32),
            pltpu.VMEM((CP, d), jnp.float32),
            pltpu.SemaphoreType.DMA,
        ])
    def gat(ys_hbm, pos_hbm, yp_hbm, pos_v, rows_v, sem):
        wid = lax.axis_index("s") * NC + lax.axis_index("c")
        base = wid * CP
        pltpu.sync_copy(pos_hbm.at[pl.ds(base, CP)], pos_v)
        pltpu.async_copy(ys_hbm.at[pos_v], rows_v, sem).wait()
        pltpu.sync_copy(rows_v, yp_hbm.at[pl.ds(base, CP)])

    return gat(ys, pos)


def _combine_body(y0_ref, y1_ref, w_ref, out_ref):
    out_ref[...] = (y0_ref[...] * w_ref[:, 0:1]
                    + y1_ref[...] * w_ref[:, 1:2])


def _combine_call(yp, w):
    P, d = yp.shape
    T = P // 2
    BMc = 256
    NTc = T // BMc
    return pl.pallas_call(
        _combine_body,
        grid=(NTc,),
        in_specs=[
            pl.BlockSpec((BMc, d), lambda i: (i, 0)),
            pl.BlockSpec((BMc, d), lambda i: (i + NTc, 0)),
            pl.BlockSpec((BMc, 2), lambda i: (i, 0)),
        ],
        out_specs=pl.BlockSpec((BMc, d), lambda i: (i, 0)),
        out_shape=jax.ShapeDtypeStruct((T, d), jnp.float32),
    )(yp, yp, w)


def kernel(x, router_w, router_b, W1, W2, W3):
    b, s, d = x.shape
    E, H, _ = W1.shape
    T = b * s
    P = 2 * T
    NPAD = P + E * BM2
    x2d = x.reshape(T, d)
    rb2 = router_b.reshape(1, E)

    NBLK = P // BM2 + E
    pos, w, eob, totb, aux = _router_call(x2d, router_w, rb2, E)
    pos_s = pos.T.reshape(P)
    eobf = eob.reshape(NBLK)
    totb1 = totb.reshape(1)

    xs = _dispatch_sc(x2d, pos_s, NPAD)

    ys = _ffn_call(xs, W1, W2, W3, eobf, totb1, NBLK)

    yp = _gather_sc(ys, pos_s)

    out2d = _combine_call(yp, w)
    return out2d.reshape(b, s, d), aux.reshape(())


# speedup vs baseline: 4.6282x; 1.4763x over previous
import functools

import jax
import numpy as np
import jax.numpy as jnp
from jax import lax
from jax.experimental import pallas as pl
from jax.experimental.pallas import tpu as pltpu
from jax.experimental.pallas import tpu_sc as plsc

BM = 256
BM2 = 256


def _router_body(x_ref, rw_ref, rb_ref,
                 pos_ref, w_ref, eob_ref, totb_ref, aux_ref,
                 eidx_s, rank_s, w_s, cnt_ref, off_s, s_ref,
                 *, E, NT, BM, BM2, NBLK):
    p = pl.program_id(0)
    i = pl.program_id(1)
    sl = pl.ds(i * BM, BM)

    @pl.when(p == 0)
    def _():
        xt = x_ref[...]
        rw = rw_ref[...]
        logits = lax.dot_general(xt, rw, (((1,), (1,)), ((), ())),
                                 preferred_element_type=jnp.float32) + rb_ref[...]
        nrm = jnp.sqrt(jnp.sum(logits * logits, axis=-1, keepdims=True))
        rl = logits / jnp.maximum(nrm, 1e-12)
        m = jnp.max(rl, axis=-1, keepdims=True)
        ex = jnp.exp(rl - m)
        probs = ex / jnp.sum(ex, axis=-1, keepdims=True)

        lane = lax.broadcasted_iota(jnp.int32, probs.shape, 1)
        m1 = jnp.max(probs, axis=-1, keepdims=True)
        i1 = jnp.min(jnp.where(probs == m1, lane, E), axis=-1, keepdims=True)
        pr2 = jnp.where(lane == i1, -jnp.inf, probs)
        m2 = jnp.max(pr2, axis=-1, keepdims=True)
        i2 = jnp.min(jnp.where(pr2 == m2, lane, E), axis=-1, keepdims=True)

        oh0 = (lane == i1).astype(jnp.float32)
        oh1 = (lane == i2).astype(jnp.float32)
        r = lax.broadcasted_iota(jnp.int32, (BM, BM), 0)
        c = lax.broadcasted_iota(jnp.int32, (BM, BM), 1)
        tri = (r > c).astype(jnp.float32)
        excl0 = lax.dot_general(tri, oh0, (((1,), (0,)), ((), ())),
                                preferred_element_type=jnp.float32)
        tot0 = jnp.sum(oh0, axis=0, keepdims=True)
        excl1 = lax.dot_general(tri, oh1, (((1,), (0,)), ((), ())),
                                preferred_element_type=jnp.float32) + tot0
        tot1 = jnp.sum(oh1, axis=0, keepdims=True)

        base = jnp.where(i == 0, jnp.zeros((1, E), jnp.float32), cnt_ref[...])
        rank0 = jnp.sum(oh0 * (excl0 + base), axis=1, keepdims=True)
        rank1 = jnp.sum(oh1 * (excl1 + base), axis=1, keepdims=True)
        newcnt = base + tot0 + tot1
        cnt_ref[...] = newcnt

        eidx_s[sl, :] = jnp.concatenate([i1, i2], axis=1)
        rank_s[sl, :] = jnp.concatenate([rank0, rank1], axis=1).astype(jnp.int32)
        w_s[sl, :] = jnp.concatenate([m1, m2], axis=1)

        partial = jnp.sum((1.0 / E - probs) ** 2)
        tot = jnp.where(i == 0, 0.0, s_ref[0]) + partial
        s_ref[0] = tot

        @pl.when(i == NT - 1)
        def _():
            cnt_i = newcnt.astype(jnp.int32)
            ntb = (cnt_i + (BM2 - 1)) // BM2
            ntb_f = ntb.astype(jnp.float32)
            rr = lax.broadcasted_iota(jnp.int32, (E, E), 0)
            cc = lax.broadcasted_iota(jnp.int32, (E, E), 1)
            ustr = (rr < cc).astype(jnp.float32)
            offb = lax.dot_general(ntb_f, ustr, (((1,), (0,)), ((), ())),
                                   preferred_element_type=jnp.float32)
            offb_i = offb.astype(jnp.int32)
            off_s[...] = offb_i * BM2
            blkio = lax.broadcasted_iota(jnp.int32, (1, NBLK), 1)
            eob = jnp.full((1, NBLK), -1, jnp.int32)
            for e in range(E):
                eob = eob + jnp.where(blkio >= offb_i[0:1, e:e + 1], 1, 0)
            eob_ref[...] = eob
            totb_ref[...] = offb_i[0:1, E - 1:E] + ntb[0:1, E - 1:E]
            aux_ref[...] = jnp.full((1, 1), tot, dtype=jnp.float32)

    @pl.when(p == 1)
    def _():
        ei = eidx_s[sl, :]
        rk = rank_s[sl, :]
        po = jnp.zeros(ei.shape, jnp.int32)
        for e in range(E):
            po = po + jnp.where(ei == e, 1, 0) * off_s[0:1, e:e + 1]
        pos_ref[...] = po + rk
        w_ref[...] = w_s[sl, :]


def _router_call(x2d, router_w, rb2, E):
    T, d = x2d.shape
    NT = T // BM
    NBLK = 2 * T // BM2 + E
    body = functools.partial(_router_body, E=E, NT=NT, BM=BM, BM2=BM2,
                             NBLK=NBLK)
    return pl.pallas_call(
        body,
        grid=(2, NT),
        in_specs=[
            pl.BlockSpec((BM, d), lambda p, i: (jnp.where(p == 0, i, NT - 1), 0)),
            pl.BlockSpec((E, d), lambda p, i: (0, 0)),
            pl.BlockSpec((1, E), lambda p, i: (0, 0)),
        ],
        out_specs=[
            pl.BlockSpec((BM, 2), lambda p, i: (jnp.where(p == 1, i, 0), 0)),
            pl.BlockSpec((BM, 2), lambda p, i: (jnp.where(p == 1, i, 0), 0)),
            pl.BlockSpec((1, NBLK), lambda p, i: (0, 0)),
            pl.BlockSpec((1, 1), lambda p, i: (0, 0)),
            pl.BlockSpec((1, 1), lambda p, i: (0, 0)),
        ],
        out_shape=[
            jax.ShapeDtypeStruct((T, 2), jnp.int32),
            jax.ShapeDtypeStruct((T, 2), jnp.float32),
            jax.ShapeDtypeStruct((1, NBLK), jnp.int32),
            jax.ShapeDtypeStruct((1, 1), jnp.int32),
            jax.ShapeDtypeStruct((1, 1), jnp.float32),
        ],
        scratch_shapes=[
            pltpu.VMEM((T, 2), jnp.int32),
            pltpu.VMEM((T, 2), jnp.int32),
            pltpu.VMEM((T, 2), jnp.float32),
            pltpu.VMEM((1, E), jnp.float32),
            pltpu.VMEM((1, E), jnp.int32),
            pltpu.SMEM((1,), jnp.float32),
        ],
        compiler_params=pltpu.CompilerParams(
            dimension_semantics=("arbitrary", "arbitrary")),
    )(x2d, router_w, rb2)


def _dispatch_sc(x2d, pos_s, NPAD):
    T, d = x2d.shape
    P = pos_s.shape[0]
    info = plsc.get_sparse_core_info()
    NC, NS = info.num_cores, info.num_subcores
    NW = NC * NS
    CP = P // NW
    mesh = plsc.VectorSubcoreMesh(core_axis_name="c", subcore_axis_name="s")

    @functools.partial(
        pl.kernel, mesh=mesh,
        out_type=jax.ShapeDtypeStruct((NPAD, d), jnp.float32),
        scratch_types=[
            pltpu.VMEM((CP,), jnp.int32),
            pltpu.VMEM((CP, d), jnp.float32),
            pltpu.SemaphoreType.DMA,
        ])
    def disp(x_hbm, pos_hbm, xs_hbm, pos_v, rows_v, sem):
        wid = lax.axis_index("s") * NC + lax.axis_index("c")
        base = wid * CP
        tok = base - jnp.where(base >= T, T, 0)
        pltpu.sync_copy(pos_hbm.at[pl.ds(base, CP)], pos_v)
        pltpu.sync_copy(x_hbm.at[pl.ds(tok, CP)], rows_v)
        pltpu.async_copy(rows_v, xs_hbm.at[pos_v], sem).wait()

    return disp(x2d, pos_s)


def _fast_sin(x):
    kf = jnp.floor(x * np.float32(1.0 / np.pi) + 0.5)
    r = x - kf * np.float32(3.140625)
    r = r - kf * np.float32(9.676535897932e-4)
    ki = kf.astype(jnp.int32)
    sign = jnp.where((ki & 1) == 0, np.float32(1.0), np.float32(-1.0))
    r2 = r * r
    s = np.float32(2.7526048e-6)
    s = s * r2 + np.float32(-1.9840988e-4)
    s = s * r2 + np.float32(8.3333310e-3)
    s = s * r2 + np.float32(-1.6666666e-1)
    s = r + r * (r2 * s)
    return sign * s


def _ffn_body(eob_ref, totb_ref, xs_ref, w1_ref, w2_ref, w3_ref, ys_ref):
    t = pl.program_id(0)

    @pl.when(t < totb_ref[0])
    def _():
        xt = xs_ref[...].astype(jnp.bfloat16)
        w1 = w1_ref[0].astype(jnp.bfloat16)
        w3 = w3_ref[0].astype(jnp.bfloat16)
        w2 = w2_ref[0].astype(jnp.bfloat16)
        dn = (((1,), (1,)), ((), ()))
        h = _fast_sin(lax.dot_general(xt, w1, dn,
                                      preferred_element_type=jnp.float32))
        h = h * lax.dot_general(xt, w3, dn,
                                preferred_element_type=jnp.float32)
        ys_ref[...] = lax.dot_general(h.astype(jnp.bfloat16), w2, dn,
                                      preferred_element_type=jnp.float32)


def _ffn_call(xs, W1, W2, W3, eob, totb, NBLK):
    NPAD, d = xs.shape
    E, H, _ = W1.shape

    def xs_map(t, eob_ref, totb_ref):
        return (jnp.minimum(t, totb_ref[0] - 1), 0)

    def w_map(t, eob_ref, totb_ref):
        return (eob_ref[jnp.minimum(t, totb_ref[0] - 1)], 0, 0)

    grid_spec = pltpu.PrefetchScalarGridSpec(
        num_scalar_prefetch=2,
        grid=(NBLK,),
        in_specs=[
            pl.BlockSpec((BM2, d), xs_map),
            pl.BlockSpec((1, H, d), w_map),
            pl.BlockSpec((1, d, H), w_map),
            pl.BlockSpec((1, H, d), w_map),
        ],
        out_specs=pl.BlockSpec((BM2, d), xs_map),
    )
    return pl.pallas_call(
        _ffn_body,
        grid_spec=grid_spec,
        out_shape=jax.ShapeDtypeStruct((NPAD, d), jnp.float32),
        compiler_params=pltpu.CompilerParams(
            dimension_semantics=("arbitrary",)),
    )(eob, totb, xs, W1, W2, W3)


def _gather_sc(ys, pos):
    NPAD, d = ys.shape
    P = pos.shape[0]
    info = plsc.get_sparse_core_info()
    NC, NS = info.num_cores, info.num_subcores
    NW = NC * NS
    CP = P // NW
    mesh = plsc.VectorSubcoreMesh(core_axis_name="c", subcore_axis_name="s")

    @functools.partial(
        pl.kernel, mesh=mesh,
        out_type=jax.ShapeDtypeStruct((P, d), jnp.float32),
        scratch_types=[
            pltpu.VMEM((CP,), jnp.int32),
            pltpu.VMEM((CP, d), jnp.float32),
            pltpu.SemaphoreType.DMA,
        ])
    def gat(ys_hbm, pos_hbm, yp_hbm, pos_v, rows_v, sem):
        wid = lax.axis_index("s") * NC + lax.axis_index("c")
        base = wid * CP
        pltpu.sync_copy(pos_hbm.at[pl.ds(base, CP)], pos_v)
        pltpu.async_copy(ys_hbm.at[pos_v], rows_v, sem).wait()
        pltpu.sync_copy(rows_v, yp_hbm.at[pl.ds(base, CP)])

    return gat(ys, pos)


def _combine_body(y0_ref, y1_ref, w_ref, out_ref):
    out_ref[...] = (y0_ref[...] * w_ref[:, 0:1]
                    + y1_ref[...] * w_ref[:, 1:2])


def _combine_call(yp, w):
    P, d = yp.shape
    T = P // 2
    BMc = 256
    NTc = T // BMc
    return pl.pallas_call(
        _combine_body,
        grid=(NTc,),
        in_specs=[
            pl.BlockSpec((BMc, d), lambda i: (i, 0)),
            pl.BlockSpec((BMc, d), lambda i: (i + NTc, 0)),
            pl.BlockSpec((BMc, 2), lambda i: (i, 0)),
        ],
        out_specs=pl.BlockSpec((BMc, d), lambda i: (i, 0)),
        out_shape=jax.ShapeDtypeStruct((T, d), jnp.float32),
    )(yp, yp, w)


def kernel(x, router_w, router_b, W1, W2, W3):
    b, s, d = x.shape
    E, H, _ = W1.shape
    T = b * s
    P = 2 * T
    NPAD = P + E * BM2
    x2d = x.reshape(T, d)
    rb2 = router_b.reshape(1, E)

    NBLK = P // BM2 + E
    pos, w, eob, totb, aux = _router_call(x2d, router_w, rb2, E)
    pos_s = pos.T.reshape(P)
    eobf = eob.reshape(NBLK)
    totb1 = totb.reshape(1)

    xs = _dispatch_sc(x2d, pos_s, NPAD)

    ys = _ffn_call(xs, W1, W2, W3, eobf, totb1, NBLK)

    yp = _gather_sc(ys, pos_s)

    out2d = _combine_call(yp, w)
    return out2d.reshape(b, s, d), aux.reshape(())


# speedup vs baseline: 4.7362x; 1.0233x over previous
import functools

import jax
import numpy as np
import jax.numpy as jnp
from jax import lax
from jax.experimental import pallas as pl
from jax.experimental.pallas import tpu as pltpu
from jax.experimental.pallas import tpu_sc as plsc

BM = 512
BM2 = 256


def _router_body(x_ref, rw_ref, rb_ref,
                 pos_ref, w_ref, eob_ref, totb_ref, aux_ref,
                 eidx_s, rank_s, w_s, cnt_ref, off_s, s_ref,
                 *, E, NT, BM, BM2, NBLK):
    p = pl.program_id(0)
    i = pl.program_id(1)
    sl = pl.ds(i * BM, BM)

    @pl.when(p == 0)
    def _():
        xt = x_ref[...]
        rw = rw_ref[...]
        logits = lax.dot_general(xt, rw, (((1,), (1,)), ((), ())),
                                 preferred_element_type=jnp.float32) + rb_ref[...]
        nrm = jnp.sqrt(jnp.sum(logits * logits, axis=-1, keepdims=True))
        rl = logits / jnp.maximum(nrm, 1e-12)
        m = jnp.max(rl, axis=-1, keepdims=True)
        ex = jnp.exp(rl - m)
        probs = ex / jnp.sum(ex, axis=-1, keepdims=True)

        lane = lax.broadcasted_iota(jnp.int32, probs.shape, 1)
        m1 = jnp.max(probs, axis=-1, keepdims=True)
        i1 = jnp.min(jnp.where(probs == m1, lane, E), axis=-1, keepdims=True)
        pr2 = jnp.where(lane == i1, -jnp.inf, probs)
        m2 = jnp.max(pr2, axis=-1, keepdims=True)
        i2 = jnp.min(jnp.where(pr2 == m2, lane, E), axis=-1, keepdims=True)

        oh0 = (lane == i1).astype(jnp.float32)
        oh1 = (lane == i2).astype(jnp.float32)
        r = lax.broadcasted_iota(jnp.int32, (BM, BM), 0)
        c = lax.broadcasted_iota(jnp.int32, (BM, BM), 1)
        tri = (r > c).astype(jnp.float32)
        excl0 = lax.dot_general(tri, oh0, (((1,), (0,)), ((), ())),
                                preferred_element_type=jnp.float32)
        tot0 = jnp.sum(oh0, axis=0, keepdims=True)
        excl1 = lax.dot_general(tri, oh1, (((1,), (0,)), ((), ())),
                                preferred_element_type=jnp.float32) + tot0
        tot1 = jnp.sum(oh1, axis=0, keepdims=True)

        base = jnp.where(i == 0, jnp.zeros((1, E), jnp.float32), cnt_ref[...])
        rank0 = jnp.sum(oh0 * (excl0 + base), axis=1, keepdims=True)
        rank1 = jnp.sum(oh1 * (excl1 + base), axis=1, keepdims=True)
        newcnt = base + tot0 + tot1
        cnt_ref[...] = newcnt

        eidx_s[sl, :] = jnp.concatenate([i1, i2], axis=1)
        rank_s[sl, :] = jnp.concatenate([rank0, rank1], axis=1).astype(jnp.int32)
        w_s[sl, :] = jnp.concatenate([m1, m2], axis=1)

        partial = jnp.sum((1.0 / E - probs) ** 2)
        tot = jnp.where(i == 0, 0.0, s_ref[0]) + partial
        s_ref[0] = tot

        @pl.when(i == NT - 1)
        def _():
            cnt_i = newcnt.astype(jnp.int32)
            ntb = (cnt_i + (BM2 - 1)) // BM2
            ntb_f = ntb.astype(jnp.float32)
            rr = lax.broadcasted_iota(jnp.int32, (E, E), 0)
            cc = lax.broadcasted_iota(jnp.int32, (E, E), 1)
            ustr = (rr < cc).astype(jnp.float32)
            offb = lax.dot_general(ntb_f, ustr, (((1,), (0,)), ((), ())),
                                   preferred_element_type=jnp.float32)
            offb_i = offb.astype(jnp.int32)
            off_s[...] = offb_i * BM2
            blkio = lax.broadcasted_iota(jnp.int32, (1, NBLK), 1)
            eob = jnp.full((1, NBLK), -1, jnp.int32)
            for e in range(E):
                eob = eob + jnp.where(blkio >= offb_i[0:1, e:e + 1], 1, 0)
            eob_ref[...] = eob
            totb_ref[...] = offb_i[0:1, E - 1:E] + ntb[0:1, E - 1:E]
            aux_ref[...] = jnp.full((1, 1), tot, dtype=jnp.float32)

    @pl.when(p == 1)
    def _():
        ei = eidx_s[sl, :]
        rk = rank_s[sl, :]
        po = jnp.zeros(ei.shape, jnp.int32)
        for e in range(E):
            po = po + jnp.where(ei == e, 1, 0) * off_s[0:1, e:e + 1]
        pos_ref[...] = po + rk
        w_ref[...] = w_s[sl, :]


def _router_call(x2d, router_w, rb2, E):
    T, d = x2d.shape
    NT = T // BM
    NBLK = 2 * T // BM2 + E
    body = functools.partial(_router_body, E=E, NT=NT, BM=BM, BM2=BM2,
                             NBLK=NBLK)
    return pl.pallas_call(
        body,
        grid=(2, NT),
        in_specs=[
            pl.BlockSpec((BM, d), lambda p, i: (jnp.where(p == 0, i, NT - 1), 0)),
            pl.BlockSpec((E, d), lambda p, i: (0, 0)),
            pl.BlockSpec((1, E), lambda p, i: (0, 0)),
        ],
        out_specs=[
            pl.BlockSpec((BM, 2), lambda p, i: (jnp.where(p == 1, i, 0), 0)),
            pl.BlockSpec((BM, 2), lambda p, i: (jnp.where(p == 1, i, 0), 0)),
            pl.BlockSpec((1, NBLK), lambda p, i: (0, 0)),
            pl.BlockSpec((1, 1), lambda p, i: (0, 0)),
            pl.BlockSpec((1, 1), lambda p, i: (0, 0)),
        ],
        out_shape=[
            jax.ShapeDtypeStruct((T, 2), jnp.int32),
            jax.ShapeDtypeStruct((T, 2), jnp.float32),
            jax.ShapeDtypeStruct((1, NBLK), jnp.int32),
            jax.ShapeDtypeStruct((1, 1), jnp.int32),
            jax.ShapeDtypeStruct((1, 1), jnp.float32),
        ],
        scratch_shapes=[
            pltpu.VMEM((T, 2), jnp.int32),
            pltpu.VMEM((T, 2), jnp.int32),
            pltpu.VMEM((T, 2), jnp.float32),
            pltpu.VMEM((1, E), jnp.float32),
            pltpu.VMEM((1, E), jnp.int32),
            pltpu.SMEM((1,), jnp.float32),
        ],
        compiler_params=pltpu.CompilerParams(
            dimension_semantics=("arbitrary", "arbitrary")),
    )(x2d, router_w, rb2)


def _dispatch_sc(x2d, pos_s, NPAD):
    T, d = x2d.shape
    P = pos_s.shape[0]
    info = plsc.get_sparse_core_info()
    NC, NS = info.num_cores, info.num_subcores
    NW = NC * NS
    CP = P // NW
    mesh = plsc.VectorSubcoreMesh(core_axis_name="c", subcore_axis_name="s")

    @functools.partial(
        pl.kernel, mesh=mesh,
        out_type=jax.ShapeDtypeStruct((NPAD, d), jnp.float32),
        scratch_types=[
            pltpu.VMEM((CP,), jnp.int32),
            pltpu.VMEM((CP, d), jnp.float32),
            pltpu.SemaphoreType.DMA,
        ])
    def disp(x_hbm, pos_hbm, xs_hbm, pos_v, rows_v, sem):
        wid = lax.axis_index("s") * NC + lax.axis_index("c")
        base = wid * CP
        tok = base - jnp.where(base >= T, T, 0)
        pltpu.sync_copy(pos_hbm.at[pl.ds(base, CP)], pos_v)
        pltpu.sync_copy(x_hbm.at[pl.ds(tok, CP)], rows_v)
        pltpu.async_copy(rows_v, xs_hbm.at[pos_v], sem).wait()

    return disp(x2d, pos_s)


def _fast_sin(x):
    kf = jnp.floor(x * np.float32(1.0 / np.pi) + 0.5)
    r = x - kf * np.float32(3.140625)
    r = r - kf * np.float32(9.676535897932e-4)
    ki = kf.astype(jnp.int32)
    sign = jnp.where((ki & 1) == 0, np.float32(1.0), np.float32(-1.0))
    r2 = r * r
    s = np.float32(2.7526048e-6)
    s = s * r2 + np.float32(-1.9840988e-4)
    s = s * r2 + np.float32(8.3333310e-3)
    s = s * r2 + np.float32(-1.6666666e-1)
    s = r + r * (r2 * s)
    return sign * s


def _ffn_body(eob_ref, totb_ref, xs_ref, w1_ref, w2_ref, w3_ref, ys_ref):
    t = pl.program_id(0)

    @pl.when(t < totb_ref[0])
    def _():
        xt = xs_ref[...].astype(jnp.bfloat16)
        w1 = w1_ref[0].astype(jnp.bfloat16)
        w3 = w3_ref[0].astype(jnp.bfloat16)
        w2 = w2_ref[0].astype(jnp.bfloat16)
        dn = (((1,), (1,)), ((), ()))
        h = _fast_sin(lax.dot_general(xt, w1, dn,
                                      preferred_element_type=jnp.float32))
        h = h * lax.dot_general(xt, w3, dn,
                                preferred_element_type=jnp.float32)
        ys_ref[...] = lax.dot_general(h.astype(jnp.bfloat16), w2, dn,
                                      preferred_element_type=jnp.float32)


def _ffn_call(xs, W1, W2, W3, eob, totb, NBLK):
    NPAD, d = xs.shape
    E, H, _ = W1.shape

    def xs_map(t, eob_ref, totb_ref):
        return (jnp.minimum(t, totb_ref[0] - 1), 0)

    def w_map(t, eob_ref, totb_ref):
        return (eob_ref[jnp.minimum(t, totb_ref[0] - 1)], 0, 0)

    grid_spec = pltpu.PrefetchScalarGridSpec(
        num_scalar_prefetch=2,
        grid=(NBLK,),
        in_specs=[
            pl.BlockSpec((BM2, d), xs_map),
            pl.BlockSpec((1, H, d), w_map),
            pl.BlockSpec((1, d, H), w_map),
            pl.BlockSpec((1, H, d), w_map),
        ],
        out_specs=pl.BlockSpec((BM2, d), xs_map),
    )
    return pl.pallas_call(
        _ffn_body,
        grid_spec=grid_spec,
        out_shape=jax.ShapeDtypeStruct((NPAD, d), jnp.float32),
        compiler_params=pltpu.CompilerParams(
            dimension_semantics=("arbitrary",)),
    )(eob, totb, xs, W1, W2, W3)


def _gather_sc(ys, pos):
    NPAD, d = ys.shape
    P = pos.shape[0]
    info = plsc.get_sparse_core_info()
    NC, NS = info.num_cores, info.num_subcores
    NW = NC * NS
    CP = P // NW
    mesh = plsc.VectorSubcoreMesh(core_axis_name="c", subcore_axis_name="s")

    @functools.partial(
        pl.kernel, mesh=mesh,
        out_type=jax.ShapeDtypeStruct((P, d), jnp.float32),
        scratch_types=[
            pltpu.VMEM((CP,), jnp.int32),
            pltpu.VMEM((CP, d), jnp.float32),
            pltpu.SemaphoreType.DMA,
        ])
    def gat(ys_hbm, pos_hbm, yp_hbm, pos_v, rows_v, sem):
        wid = lax.axis_index("s") * NC + lax.axis_index("c")
        base = wid * CP
        pltpu.sync_copy(pos_hbm.at[pl.ds(base, CP)], pos_v)
        pltpu.async_copy(ys_hbm.at[pos_v], rows_v, sem).wait()
        pltpu.sync_copy(rows_v, yp_hbm.at[pl.ds(base, CP)])

    return gat(ys, pos)


def _combine_body(y0_ref, y1_ref, w_ref, out_ref):
    out_ref[...] = (y0_ref[...] * w_ref[:, 0:1]
                    + y1_ref[...] * w_ref[:, 1:2])


def _combine_call(yp, w):
    P, d = yp.shape
    T = P // 2
    BMc = 256
    NTc = T // BMc
    return pl.pallas_call(
        _combine_body,
        grid=(NTc,),
        in_specs=[
            pl.BlockSpec((BMc, d), lambda i: (i, 0)),
            pl.BlockSpec((BMc, d), lambda i: (i + NTc, 0)),
            pl.BlockSpec((BMc, 2), lambda i: (i, 0)),
        ],
        out_specs=pl.BlockSpec((BMc, d), lambda i: (i, 0)),
        out_shape=jax.ShapeDtypeStruct((T, d), jnp.float32),
    )(yp, yp, w)


def kernel(x, router_w, router_b, W1, W2, W3):
    b, s, d = x.shape
    E, H, _ = W1.shape
    T = b * s
    P = 2 * T
    NPAD = P + E * BM2
    x2d = x.reshape(T, d)
    rb2 = router_b.reshape(1, E)

    NBLK = P // BM2 + E
    pos, w, eob, totb, aux = _router_call(x2d, router_w, rb2, E)
    pos_s = pos.T.reshape(P)
    eobf = eob.reshape(NBLK)
    totb1 = totb.reshape(1)

    xs = _dispatch_sc(x2d, pos_s, NPAD)
    ys = _ffn_call(xs, W1, W2, W3, eobf, totb1, NBLK)
    yp = _gather_sc(ys, pos_s)
    out2d = _combine_call(yp, w)
    return out2d.reshape(b, s, d), aux.reshape(())
